# baseline (speedup 1.0000x reference)
import functools

import jax
import jax.numpy as jnp
from jax import lax
from jax.experimental import pallas as pl
from jax.experimental.pallas import tpu as pltpu
from jax.experimental.pallas import tpu_sc as plsc

B = 16384
D = 64
_INFO = plsc.get_sparse_core_info()
_NC = _INFO.num_cores
_NS = _INFO.num_subcores
_NW = _NC * _NS
_BPW = B // _NW
_CH = 128
_NCH = _BPW // _CH
_L = 16


def _lane_perm(x, idx):
    dnums = lax.GatherDimensionNumbers(
        offset_dims=(), collapsed_slice_dims=(0,), start_index_map=(0,))
    return lax.gather(x, idx[:, None], dnums, (1,),
                      mode=lax.GatherScatterMode.PROMISE_IN_BOUNDS)


def _body(mi_hbm, ui_hbm, mw_hbm, uw_hbm, mb_hbm, ub_hbm, out_hbm,
          mi_v, ui_v, mrows, urows, mb_v, ub_v, out_v, sem):
    wid = lax.axis_index("s") * _NC + lax.axis_index("c")
    base = wid * _BPW

    for c in range(_NCH):
        pltpu.sync_copy(mi_hbm.at[pl.ds(base + c * _CH, _CH)], mi_v.at[c])
        pltpu.sync_copy(ui_hbm.at[pl.ds(base + c * _CH, _CH)], ui_v.at[c])

    copies = []
    for c in range(_NCH):
        sl = pl.ds(c * _CH, _CH)
        copies.append(pltpu.async_copy(mw_hbm.at[mi_v.at[c]], mrows.at[sl], sem))
        copies.append(pltpu.async_copy(uw_hbm.at[ui_v.at[c]], urows.at[sl], sem))
        copies.append(pltpu.async_copy(mb_hbm.at[mi_v.at[c]], mb_v.at[sl], sem))
        copies.append(pltpu.async_copy(ub_hbm.at[ui_v.at[c]], ub_v.at[sl], sem))
    for cp in copies:
        cp.wait()

    iota = lax.iota(jnp.int32, _L)

    def group(g, _):
        g16 = g * _L
        tot = jnp.zeros((_L,), jnp.float32)
        for r in range(_L):
            row = g16 + r
            acc = None
            for k in range(D // _L):
                m = mrows[row, pl.ds(k * _L, _L)]
                u = urows[row, pl.ds(k * _L, _L)]
                prod = m * u
                acc = prod if acc is None else acc + prod
            for sh in (8, 4, 2, 1):
                acc = acc + _lane_perm(acc, iota ^ sh)
            tot = jnp.where(iota == r, acc, tot)
        x = tot + mb_v[pl.ds(g16, _L)] + ub_v[pl.ds(g16, _L)]
        y = 1.0 / (1.0 + jnp.exp(-x))
        out_v[pl.ds(g16, _L)] = y
        return _

    lax.fori_loop(0, _BPW // _L, group, 0)
    pltpu.sync_copy(out_v, out_hbm.at[pl.ds(base, _BPW)])


@jax.jit
def _run(movie_idx, user_idx, movies_weights, users_weights,
         movies_biases, users_biases):
    mesh = plsc.VectorSubcoreMesh(core_axis_name="c", subcore_axis_name="s")
    f = pl.kernel(
        _body,
        mesh=mesh,
        compiler_params=pltpu.CompilerParams(use_tc_tiling_on_sc=False),
        out_type=jax.ShapeDtypeStruct((B,), jnp.float32),
        scratch_types=[
            pltpu.VMEM((_NCH, _CH), jnp.int32),
            pltpu.VMEM((_NCH, _CH), jnp.int32),
            pltpu.VMEM((_BPW, D), jnp.float32),
            pltpu.VMEM((_BPW, D), jnp.float32),
            pltpu.VMEM((_BPW,), jnp.float32),
            pltpu.VMEM((_BPW,), jnp.float32),
            pltpu.VMEM((_BPW,), jnp.float32),
            pltpu.SemaphoreType.DMA,
        ],
    )
    return f(movie_idx, user_idx, movies_weights, users_weights,
             movies_biases, users_biases)


def kernel(movie_idx, user_idx, movies_weights, users_weights,
           movies_biases, users_biases):
    return _run(movie_idx.astype(jnp.int32), user_idx.astype(jnp.int32),
                movies_weights, users_weights, movies_biases, users_biases)


# speedup vs baseline: 1.6423x; 1.6423x over previous
import jax
import jax.numpy as jnp
from jax import lax
from jax.experimental import pallas as pl
from jax.experimental.pallas import tpu as pltpu
from jax.experimental.pallas import tpu_sc as plsc

B = 16384
D = 64
_INFO = plsc.get_sparse_core_info()
_NC = _INFO.num_cores
_NS = _INFO.num_subcores
_NW = _NC * _NS
_BPW = B // _NW
_L = 16
_C = 128
_NCH = _BPW // _C
_G = _C // _L


def _lane_perm(x, idx):
    dnums = lax.GatherDimensionNumbers(
        offset_dims=(), collapsed_slice_dims=(0,), start_index_map=(0,))
    return lax.gather(x, idx[:, None], dnums, (1,),
                      mode=lax.GatherScatterMode.PROMISE_IN_BOUNDS)


def _body(mi_hbm, ui_hbm, mw_hbm, uw_hbm, mb_hbm, ub_hbm, out_hbm,
          mi_v, ui_v, mrA, mrB, urA, urB, mb_v, ub_v, out_v,
          semA, semB, bsem):
    wid = lax.axis_index("s") * _NC + lax.axis_index("c")
    base = wid * _BPW

    pltpu.sync_copy(mi_hbm.at[pl.ds(base, _BPW)], mi_v)
    pltpu.sync_copy(ui_hbm.at[pl.ds(base, _BPW)], ui_v)

    def bias_issue(g, carry):
        gl = g * _L
        miv = mi_v[pl.ds(gl, _L)]
        uiv = ui_v[pl.ds(gl, _L)]
        for r in range(_L):
            i = gl + r
            mo8 = pl.multiple_of((miv[r] // 8) * 8, 8)
            uo8 = pl.multiple_of((uiv[r] // 8) * 8, 8)
            pltpu.async_copy(mb_hbm.at[pl.ds(mo8, 8)],
                             mb_v.at[pl.ds(i * 8, 8)], bsem)
            pltpu.async_copy(ub_hbm.at[pl.ds(uo8, 8)],
                             ub_v.at[pl.ds(i * 8, 8)], bsem)
        return carry

    lax.fori_loop(0, _BPW // _L, bias_issue, 0)

    def fire(c, mr, ur, sem):
        def g(gg, carry):
            gl = c * _C + gg * _L
            miv = mi_v[pl.ds(gl, _L)]
            uiv = ui_v[pl.ds(gl, _L)]
            for r in range(_L):
                i = gg * _L + r
                pltpu.async_copy(mw_hbm.at[miv[r]], mr.at[i], sem)
                pltpu.async_copy(uw_hbm.at[uiv[r]], ur.at[i], sem)
            return carry

        lax.fori_loop(0, _G, g, 0)

    def drain(mr, ur, sem):
        pltpu.make_async_copy(mw_hbm.at[pl.ds(0, _C)], mr, sem).wait()
        pltpu.make_async_copy(uw_hbm.at[pl.ds(0, _C)], ur, sem).wait()

    iota = lax.iota(jnp.int32, _L)

    def compute(c, mr, ur):
        def g(gg, carry):
            gl = c * _C + gg * _L
            miv = mi_v[pl.ds(gl, _L)]
            uiv = ui_v[pl.ds(gl, _L)]
            tot = jnp.zeros((_L,), jnp.float32)
            for r in range(_L):
                i = gl + r
                li = gg * _L + r
                acc = None
                for k in range(D // _L):
                    m = mr[li, pl.ds(k * _L, _L)]
                    u = ur[li, pl.ds(k * _L, _L)]
                    prod = m * u
                    acc = prod if acc is None else acc + prod
                mlane = lax.rem(miv[r], 8)
                ulane = lax.rem(uiv[r], 8)
                mb16 = mb_v[pl.ds(i * 8, _L)]
                ub16 = ub_v[pl.ds(i * 8, _L)]
                acc = acc + jnp.where(iota == mlane, mb16, 0.0)
                acc = acc + jnp.where(iota == ulane, ub16, 0.0)
                for sh in (8, 4, 2, 1):
                    acc = acc + _lane_perm(acc, iota ^ sh)
                tot = jnp.where(iota == r, acc, tot)
            y = 1.0 / (1.0 + jnp.exp(-tot))
            out_v[pl.ds(gl, _L)] = y
            return carry

        lax.fori_loop(0, _G, g, 0)

    pltpu.make_async_copy(mb_hbm.at[pl.ds(0, _BPW * 8)],
                          mb_v.at[pl.ds(0, _BPW * 8)], bsem).wait()
    pltpu.make_async_copy(mb_hbm.at[pl.ds(0, _BPW * 8)],
                          ub_v.at[pl.ds(0, _BPW * 8)], bsem).wait()

    fire(0, mrA, urA, semA)

    def pairs(j, carry):
        c0 = j * 2
        fire(c0 + 1, mrB, urB, semB)
        drain(mrA, urA, semA)
        compute(c0, mrA, urA)

        @pl.when(c0 + 2 < _NCH)
        def _():
            fire(c0 + 2, mrA, urA, semA)

        drain(mrB, urB, semB)
        compute(c0 + 1, mrB, urB)
        return carry

    lax.fori_loop(0, _NCH // 2, pairs, 0)
    pltpu.sync_copy(out_v, out_hbm.at[pl.ds(base, _BPW)])


@jax.jit
def _run(movie_idx, user_idx, movies_weights, users_weights,
         movies_biases, users_biases):
    mesh = plsc.VectorSubcoreMesh(core_axis_name="c", subcore_axis_name="s")
    f = pl.kernel(
        _body,
        mesh=mesh,
        out_type=jax.ShapeDtypeStruct((B,), jnp.float32),
        scratch_types=[
            pltpu.VMEM((_BPW,), jnp.int32),
            pltpu.VMEM((_BPW,), jnp.int32),
            pltpu.VMEM((_C, D), jnp.float32),
            pltpu.VMEM((_C, D), jnp.float32),
            pltpu.VMEM((_C, D), jnp.float32),
            pltpu.VMEM((_C, D), jnp.float32),
            pltpu.VMEM((_BPW * 8 + _L,), jnp.float32),
            pltpu.VMEM((_BPW * 8 + _L,), jnp.float32),
            pltpu.VMEM((_BPW,), jnp.float32),
            pltpu.SemaphoreType.DMA,
            pltpu.SemaphoreType.DMA,
            pltpu.SemaphoreType.DMA,
        ],
    )
    return f(movie_idx, user_idx, movies_weights, users_weights,
             movies_biases, users_biases)


def kernel(movie_idx, user_idx, movies_weights, users_weights,
           movies_biases, users_biases):
    return _run(movie_idx.astype(jnp.int32), user_idx.astype(jnp.int32),
                movies_weights, users_weights, movies_biases, users_biases)
